# baseline (speedup 1.0000x reference)
import functools

import jax
import jax.numpy as jnp
from jax import lax
from jax.experimental import pallas as pl
from jax.experimental.pallas import tpu as pltpu
from jax.experimental.pallas import tpu_sc as plsc

B = 4096
F = 64
FD = 32
K = 2048
N = 100000
NC, NS = 2, 16
NW = NC * NS
BPW = B // NW
L = 16
EBC = 64


@functools.cache
def _make_sc_gather():
    mesh = plsc.VectorSubcoreMesh(core_axis_name="c", subcore_axis_name="s",
                                  num_cores=NC, num_subcores=NS)

    @functools.partial(
        pl.kernel,
        out_type=(
            jax.ShapeDtypeStruct((F, B), jnp.float32),
            jax.ShapeDtypeStruct((F, B), jnp.float32),
            jax.ShapeDtypeStruct((FD, B), jnp.float32),
            jax.ShapeDtypeStruct((B,), jnp.float32),
        ),
        mesh=mesh,
        scratch_types=[
            pltpu.VMEM((BPW,), jnp.int32),
            pltpu.VMEM((BPW,), jnp.int32),
            pltpu.VMEM((F, BPW), jnp.int32),
            pltpu.VMEM((F, BPW), jnp.int32),
            pltpu.VMEM((F, BPW), jnp.float32),
            pltpu.VMEM((F, BPW), jnp.float32),
            pltpu.VMEM((FD, BPW), jnp.float32),
            pltpu.VMEM((BPW,), jnp.float32),
            pltpu.SemaphoreType.DMA,
        ],
        compiler_params=pltpu.CompilerParams(use_tc_tiling_on_sc=False),
    )
    def _sc_gather(user_hbm, item_hbm, guf_hbm, gif_hbm, tuf_hbm, bi_hbm,
                   gut_out, git_out, tut_out, bi_out,
                   uidx_v, iidx_v, idxu_v, idxi_v,
                   gut_v, git_v, tut_v, bi_v, sem):
        wid = lax.axis_index("s") * NC + lax.axis_index("c")
        base = wid * BPW
        pltpu.sync_copy(user_hbm.at[pl.ds(base, BPW)], uidx_v)
        pltpu.sync_copy(item_hbm.at[pl.ds(base, BPW)], iidx_v)
        for c in range(BPW // L):
            vu = uidx_v[pl.ds(c * L, L)]
            vi = iidx_v[pl.ds(c * L, L)]
            for f in range(F):
                idxu_v[f, pl.ds(c * L, L)] = vu + f * N
                idxi_v[f, pl.ds(c * L, L)] = vi + f * N
        copies = []
        for f in range(F):
            copies.append(pltpu.async_copy(
                guf_hbm.at[idxu_v.at[f]], gut_v.at[f], sem))
            copies.append(pltpu.async_copy(
                gif_hbm.at[idxi_v.at[f]], git_v.at[f], sem))
        for f in range(FD):
            copies.append(pltpu.async_copy(
                tuf_hbm.at[idxu_v.at[f]], tut_v.at[f], sem))
        copies.append(pltpu.async_copy(bi_hbm.at[iidx_v], bi_v, sem))
        for c in copies:
            c.wait()
        pltpu.sync_copy(gut_v, gut_out.at[:, pl.ds(base, BPW)])
        pltpu.sync_copy(git_v, git_out.at[:, pl.ds(base, BPW)])
        pltpu.sync_copy(tut_v, tut_out.at[:, pl.ds(base, BPW)])
        pltpu.sync_copy(bi_v, bi_out.at[pl.ds(base, BPW)])

    return _sc_gather


BLK = 512
GRID = B // BLK


def _tc_body(feat_ref, eb_ref, gut_ref, git_ref, tut_ref, bi_ref,
             xui_ref, feat_out):
    feb = jnp.dot(feat_ref[...], eb_ref[...],
                  preferred_element_type=jnp.float32)
    febt = feb.T
    xui_ref[0, 0, :] = (bi_ref[0, 0, :]
                        + jnp.sum(gut_ref[...] * git_ref[...], axis=0)
                        + jnp.sum(tut_ref[...] * febt[:FD, :], axis=0)
                        + febt[FD, :])
    feat_out[...] = feat_ref[...]


_tc_fused = pl.pallas_call(
    _tc_body,
    grid=(GRID,),
    in_specs=[
        pl.BlockSpec((BLK, K), lambda i: (i, 0)),
        pl.BlockSpec((K, EBC), lambda i: (0, 0)),
        pl.BlockSpec((F, BLK), lambda i: (0, i)),
        pl.BlockSpec((F, BLK), lambda i: (0, i)),
        pl.BlockSpec((FD, BLK), lambda i: (0, i)),
        pl.BlockSpec((1, 1, BLK), lambda i: (i, 0, 0)),
    ],
    out_specs=[
        pl.BlockSpec((1, 1, BLK), lambda i: (i, 0, 0)),
        pl.BlockSpec((BLK, K), lambda i: (i, 0)),
    ],
    out_shape=[
        jax.ShapeDtypeStruct((GRID, 1, BLK), jnp.float32),
        jax.ShapeDtypeStruct((B, K), jnp.float32),
    ],
    compiler_params=pltpu.CompilerParams(
        dimension_semantics=("arbitrary",)),
)


def kernel(user, item, feature_i, Bi, Gu, Gi, Bp, Tu, E):
    gut, git, tut, beta_i = _make_sc_gather()(
        user, item,
        Gu.T.reshape(-1), Gi.T.reshape(-1), Tu.T.reshape(-1), Bi)
    eb = jnp.concatenate(
        [E, Bp, jnp.zeros((K, EBC - FD - 1), jnp.float32)], axis=1)
    xui, feat_out = _tc_fused(feature_i, eb, gut, git, tut,
                              beta_i.reshape(GRID, 1, BLK))
    return (xui.reshape(B), gut.T, git.T, feat_out, tut.T, beta_i)


# speedup vs baseline: 1.0638x; 1.0638x over previous
import functools

import jax
import jax.numpy as jnp
from jax import lax
from jax.experimental import pallas as pl
from jax.experimental.pallas import tpu as pltpu
from jax.experimental.pallas import tpu_sc as plsc

B = 4096
F = 64
FD = 32
K = 2048
NC, NS = 2, 16
NW = NC * NS
BPW = B // NW
L = 16


@functools.cache
def _make_sc_gather():
    mesh = plsc.VectorSubcoreMesh(core_axis_name="c", subcore_axis_name="s",
                                  num_cores=NC, num_subcores=NS)

    @functools.partial(
        pl.kernel,
        out_type=(
            jax.ShapeDtypeStruct((B, F), jnp.float32),
            jax.ShapeDtypeStruct((B, F), jnp.float32),
            jax.ShapeDtypeStruct((B, FD), jnp.float32),
            jax.ShapeDtypeStruct((B,), jnp.float32),
        ),
        mesh=mesh,
        scratch_types=[
            pltpu.VMEM((BPW,), jnp.int32),
            pltpu.VMEM((BPW,), jnp.int32),
            pltpu.VMEM((BPW, F), jnp.float32),
            pltpu.VMEM((BPW, F), jnp.float32),
            pltpu.VMEM((BPW, FD), jnp.float32),
            pltpu.VMEM((BPW * 8,), jnp.float32),
            pltpu.VMEM((BPW,), jnp.float32),
            pltpu.SemaphoreType.DMA,
        ],
        compiler_params=pltpu.CompilerParams(needs_layout_passes=False),
    )
    def _sc_gather(user_hbm, item_hbm, gu_hbm, gi_hbm, tu_hbm, bi_hbm,
                   gu_out, gi_out, tu_out, bi_out,
                   uidx_v, iidx_v, gu_v, gi_v, tu_v, bi_stage, bi_v, sem):
        wid = lax.axis_index("s") * NC + lax.axis_index("c")
        base = wid * BPW
        pltpu.sync_copy(user_hbm.at[pl.ds(base, BPW)], uidx_v)
        pltpu.sync_copy(item_hbm.at[pl.ds(base, BPW)], iidx_v)
        for c in range(BPW // L):
            vu = uidx_v[pl.ds(c * L, L)]
            vi = iidx_v[pl.ds(c * L, L)]
            for l in range(L):
                u = vu[l]
                it = vi[l]
                r = c * L + l
                pltpu.async_copy(gu_hbm.at[u], gu_v.at[r], sem)
                pltpu.async_copy(tu_hbm.at[u], tu_v.at[r], sem)
                pltpu.async_copy(gi_hbm.at[it], gi_v.at[r], sem)
                off = pl.multiple_of((it // 8) * 8, 8)
                pltpu.async_copy(bi_hbm.at[pl.ds(off, 8)],
                                 bi_stage.at[pl.ds(r * 8, 8)], sem)
        pltpu.make_async_copy(gu_hbm.at[pl.ds(0, BPW)], gu_v, sem).wait()
        pltpu.make_async_copy(tu_hbm.at[pl.ds(0, BPW)], tu_v, sem).wait()
        pltpu.make_async_copy(gi_hbm.at[pl.ds(0, BPW)], gi_v, sem).wait()
        pltpu.make_async_copy(bi_hbm.at[pl.ds(0, BPW * 8)],
                              bi_stage, sem).wait()
        for c in range(BPW // L):
            vi = iidx_v[pl.ds(c * L, L)]
            rows = lax.iota(jnp.int32, L) + c * L
            flat = rows * 8 + lax.rem(vi, 8)
            bi_v[pl.ds(c * L, L)] = plsc.load_gather(bi_stage, [flat])
        pltpu.sync_copy(gu_v, gu_out.at[pl.ds(base, BPW)])
        pltpu.sync_copy(gi_v, gi_out.at[pl.ds(base, BPW)])
        pltpu.sync_copy(tu_v, tu_out.at[pl.ds(base, BPW)])
        pltpu.sync_copy(bi_v, bi_out.at[pl.ds(base, BPW)])

    return _sc_gather


BLK = 512
GRID = B // BLK


def _tc_body(feat_ref, eb_ref, gu_ref, gi_ref, tu_ref, bi_ref,
             xui_ref, feat_out, gut_out, git_out, tut_out):
    feb = jnp.dot(feat_ref[...], eb_ref[...],
                  preferred_element_type=jnp.float32)
    gu = gu_ref[...]
    gi = gi_ref[...]
    tu = tu_ref[...]
    xui_ref[0, 0, :] = (bi_ref[0, 0, :]
                        + jnp.sum(gu * gi, axis=1)
                        + jnp.sum(tu * feb[:, :FD], axis=1)
                        + feb[:, FD])
    feat_out[...] = feat_ref[...]
    gut_out[...] = gu.T
    git_out[...] = gi.T
    tut_out[...] = tu.T


_tc_fused = pl.pallas_call(
    _tc_body,
    grid=(GRID,),
    in_specs=[
        pl.BlockSpec((BLK, K), lambda i: (i, 0)),
        pl.BlockSpec((K, FD + 1), lambda i: (0, 0)),
        pl.BlockSpec((BLK, F), lambda i: (i, 0)),
        pl.BlockSpec((BLK, F), lambda i: (i, 0)),
        pl.BlockSpec((BLK, FD), lambda i: (i, 0)),
        pl.BlockSpec((1, 1, BLK), lambda i: (i, 0, 0)),
    ],
    out_specs=[
        pl.BlockSpec((1, 1, BLK), lambda i: (i, 0, 0)),
        pl.BlockSpec((BLK, K), lambda i: (i, 0)),
        pl.BlockSpec((F, BLK), lambda i: (0, i)),
        pl.BlockSpec((F, BLK), lambda i: (0, i)),
        pl.BlockSpec((FD, BLK), lambda i: (0, i)),
    ],
    out_shape=[
        jax.ShapeDtypeStruct((GRID, 1, BLK), jnp.float32),
        jax.ShapeDtypeStruct((B, K), jnp.float32),
        jax.ShapeDtypeStruct((F, B), jnp.float32),
        jax.ShapeDtypeStruct((F, B), jnp.float32),
        jax.ShapeDtypeStruct((FD, B), jnp.float32),
    ],
    compiler_params=pltpu.CompilerParams(
        dimension_semantics=("arbitrary",)),
)


def kernel(user, item, feature_i, Bi, Gu, Gi, Bp, Tu, E):
    gamma_u, gamma_i, theta_u, beta_i = _make_sc_gather()(
        user, item, Gu, Gi, Tu, Bi)
    eb = jnp.concatenate([E, Bp], axis=1)
    xui, feat_out, gut, git, tut = _tc_fused(
        feature_i, eb, gamma_u, gamma_i, theta_u,
        beta_i.reshape(GRID, 1, BLK))
    return (xui.reshape(B), gut.T, git.T, feat_out, tut.T, beta_i)
